# baseline (speedup 1.0000x reference)
import functools

import jax
import jax.numpy as jnp
import numpy as np
from jax import lax
from jax.experimental import pallas as pl
from jax.experimental.pallas import tpu as pltpu
from jax.experimental.pallas import tpu_sc as plsc

N_NEIGHBOR = 8
DIM = 32
N_OLD = 32
HIST_PAD = 56
NEG = np.float32(-9e15)
_Z = np.int32(0)


def _sc_mesh():
    return plsc.VectorSubcoreMesh(core_axis_name="c", subcore_axis_name="s")


def _wid():
    info = plsc.get_sparse_core_info()
    return (lax.axis_index("s") * np.int32(info.num_cores)
            + lax.axis_index("c"))


@functools.lru_cache(maxsize=None)
def _make_row_gather_kernel(n_idx):
    info = plsc.get_sparse_core_info()
    nw = info.num_cores * info.num_subcores
    b_per_w = n_idx // nw
    n_chunks = b_per_w // 128

    @functools.partial(
        pl.kernel, mesh=_sc_mesh(),
        compiler_params=pltpu.CompilerParams(use_tc_tiling_on_sc=False),
        out_type=jax.ShapeDtypeStruct((n_idx, DIM), jnp.float32),
        scratch_types=[
            pltpu.VMEM((b_per_w,), jnp.int32),
            pltpu.VMEM((b_per_w, DIM), jnp.float32),
            pltpu.SemaphoreType.DMA,
        ],
    )
    def k(table_hbm, idx_hbm, out_hbm, idx_v, ebuf, sem):
        base = _wid() * np.int32(b_per_w)
        pltpu.sync_copy(idx_hbm.at[pl.ds(base, b_per_w)], idx_v)
        copies = []
        for c in range(n_chunks):
            copies.append(pltpu.async_copy(
                table_hbm.at[idx_v.at[pl.ds(c * 128, 128)]],
                ebuf.at[pl.ds(c * 128, 128)], sem))
        for c in copies:
            c.wait()
        pltpu.sync_copy(ebuf, out_hbm.at[pl.ds(base, b_per_w)])

    return k


@functools.lru_cache(maxsize=None)
def _make_hop_gather_kernel(n_seed, with_seed_emb, wave_rows, clicked_n=0):
    info = plsc.get_sparse_core_info()
    nw = info.num_cores * info.num_subcores
    s_per_w = n_seed // nw
    rows_w = s_per_w * N_OLD
    adj_chunk = min(s_per_w, 128)
    adj_chunks = s_per_w // adj_chunk
    n_waves = max(1, rows_w // wave_rows)
    wrows = rows_w // n_waves
    w_chunks = wrows // 128

    outs = [
        jax.ShapeDtypeStruct((n_seed, N_OLD), jnp.int32),
        jax.ShapeDtypeStruct((n_seed, N_OLD), jnp.int32),
        jax.ShapeDtypeStruct((n_seed * N_OLD, DIM), jnp.float32),
    ]
    scratch = [
        pltpu.VMEM((s_per_w,), jnp.int32),
        pltpu.VMEM((s_per_w, N_OLD), jnp.int32),
        pltpu.VMEM((s_per_w, N_OLD), jnp.int32),
        pltpu.VMEM((wrows, DIM), jnp.float32),
        pltpu.SemaphoreType.DMA,
    ]
    if with_seed_emb:
        outs.append(jax.ShapeDtypeStruct((n_seed, DIM), jnp.float32))
        scratch.insert(3, pltpu.VMEM((s_per_w, DIM), jnp.float32))
    cl_per_w = clicked_n // nw
    cl_chunks = cl_per_w // 128
    if clicked_n:
        outs.append(jax.ShapeDtypeStruct((clicked_n, DIM), jnp.float32))
        scratch.append(pltpu.VMEM((cl_per_w,), jnp.int32))
        scratch.append(pltpu.VMEM((cl_per_w, DIM), jnp.float32))

    @functools.partial(
        pl.kernel, mesh=_sc_mesh(),
        compiler_params=pltpu.CompilerParams(use_tc_tiling_on_sc=False),
        out_type=tuple(outs), scratch_types=scratch,
    )
    def k(seed_hbm, adj_e_hbm, adj_r_hbm, emb_hbm, *rest):
        cl_idx_hbm = None
        if clicked_n:
            cl_idx_hbm, *refs = rest
        else:
            refs = rest
        cl_out = idx_cl = cl_buf = None
        if with_seed_emb and clicked_n:
            (ne_out, nr_out, nee_out, se_out, cl_out,
             idx_v, ne_v, nr_v, se_v, ebuf, sem, idx_cl, cl_buf) = refs
        elif with_seed_emb:
            (ne_out, nr_out, nee_out, se_out,
             idx_v, ne_v, nr_v, se_v, ebuf, sem) = refs
        else:
            (ne_out, nr_out, nee_out,
             idx_v, ne_v, nr_v, ebuf, sem) = refs
        w = _wid()
        base = w * np.int32(s_per_w)
        pltpu.sync_copy(seed_hbm.at[pl.ds(base, s_per_w)], idx_v)
        if clicked_n:
            cl_base = w * np.int32(cl_per_w)
            pltpu.sync_copy(cl_idx_hbm.at[pl.ds(cl_base, cl_per_w)], idx_cl)
        copies = []
        if clicked_n:
            for c in range(cl_chunks):
                copies.append(pltpu.async_copy(
                    emb_hbm.at[idx_cl.at[pl.ds(c * 128, 128)]],
                    cl_buf.at[pl.ds(c * 128, 128)], sem))
        for c in range(adj_chunks):
            sl = pl.ds(c * adj_chunk, adj_chunk)
            copies.append(pltpu.async_copy(
                adj_e_hbm.at[idx_v.at[sl]], ne_v.at[sl], sem))
            copies.append(pltpu.async_copy(
                adj_r_hbm.at[idx_v.at[sl]], nr_v.at[sl], sem))
        if with_seed_emb:
            copies.append(pltpu.async_copy(emb_hbm.at[idx_v], se_v, sem))
        for c in copies:
            c.wait()
        pltpu.sync_copy(ne_v, ne_out.at[pl.ds(base, s_per_w)])
        pltpu.sync_copy(nr_v, nr_out.at[pl.ds(base, s_per_w)])
        if with_seed_emb:
            pltpu.sync_copy(se_v, se_out.at[pl.ds(base, s_per_w)])
        if clicked_n:
            pltpu.sync_copy(cl_buf, cl_out.at[pl.ds(cl_base, cl_per_w)])

        nee_base = w * np.int32(rows_w)
        seeds_per_wave = wrows // N_OLD
        for v in range(n_waves):
            copies = []
            for s in range(seeds_per_wave):
                copies.append(pltpu.async_copy(
                    emb_hbm.at[ne_v.at[np.int32(v * seeds_per_wave + s)]],
                    ebuf.at[pl.ds(s * N_OLD, N_OLD)], sem))
            for c in copies:
                c.wait()
            pltpu.sync_copy(
                ebuf, nee_out.at[pl.ds(nee_base + np.int32(v * wrows),
                                       wrows)])

    return k


def _hop_core(ue, nee_ref, nrf_ref, nef_ref, w_ref,
              su_out, neetop_out, netop_out):
    br = ue.shape[0]
    x = nee_ref[...]
    nrf = nrf_ref[...]
    for r in range(4):
        wr = w_ref[r]
        t = jnp.tanh(lax.dot_general(
            x, wr, (((1,), (1,)), ((), ())),
            preferred_element_type=jnp.float32))
        x = jnp.where(nrf == np.float32(r), t, x)
    neet = x.reshape(br, N_OLD, DIM)
    su = jnp.sum(neet * ue[:, None, :], axis=-1) * (1.0 / DIM)
    su = jnp.where(su == 0.0, NEG, su)
    m = jnp.max(su, axis=-1, keepdims=True)
    e = jnp.exp(su - m)
    su = e / jnp.sum(e, axis=-1, keepdims=True)

    iota = lax.broadcasted_iota(jnp.int32, (br, N_OLD), 1).astype(jnp.float32)
    nef = nef_ref[...]
    cur = su
    vals, ids, embs = [], [], []
    for _ in range(N_NEIGHBOR):
        mk = jnp.max(cur, axis=-1, keepdims=True)
        cand = jnp.where(cur == mk, iota, np.float32(N_OLD))
        sel = jnp.min(cand, axis=-1, keepdims=True)
        onehot = iota == sel
        ohf = onehot.astype(jnp.float32)
        vals.append(jnp.sum(su * ohf, -1, keepdims=True))
        ids.append(jnp.sum(nef * ohf, -1, keepdims=True))
        embs.append(jnp.sum(neet * ohf[:, :, None], axis=1)
                    .reshape(br, 1, DIM))
        cur = jnp.where(onehot, NEG, cur)

    st = jnp.concatenate(vals, axis=-1)
    st = jnp.where(st == 0.0, NEG, st)
    m2 = jnp.max(st, axis=-1, keepdims=True)
    e2 = jnp.exp(st - m2)
    su_out[...] = e2 / jnp.sum(e2, axis=-1, keepdims=True)
    netop_out[...] = jnp.concatenate(ids, axis=-1).astype(jnp.int32)
    neetop_out[...] = jnp.concatenate(embs, axis=1)


def _hop1_body(cl_ref, inv_ref, nee_ref, nrf_ref, nef_ref, w_ref,
               ue_out, su_out, neetop_out, netop_out):
    ue = jnp.sum(cl_ref[...], axis=1) * inv_ref[...]
    ue_out[...] = ue
    _hop_core(ue, nee_ref, nrf_ref, nef_ref, w_ref,
              su_out, neetop_out, netop_out)


def _hop2_body(ue_ref, nee_ref, nrf_ref, nef_ref, w_ref,
               su_out, neetop_out, netop_out):
    _hop_core(ue_ref[...], nee_ref, nrf_ref, nef_ref, w_ref,
              su_out, neetop_out, netop_out)


def _run_hop1(cl_emb, inv_b, nee_flat, nr_flat_f, ne_f, w_rel, br):
    rows = inv_b.shape[0]
    grid = (rows // br,)
    return pl.pallas_call(
        _hop1_body,
        grid=grid,
        in_specs=[
            pl.BlockSpec((br, HIST_PAD, DIM), lambda i: (i, _Z, _Z)),
            pl.BlockSpec((br, DIM), lambda i: (i, _Z)),
            pl.BlockSpec((br * N_OLD, DIM), lambda i: (i, _Z)),
            pl.BlockSpec((br * N_OLD, 1), lambda i: (i, _Z)),
            pl.BlockSpec((br, N_OLD), lambda i: (i, _Z)),
            pl.BlockSpec((4, DIM, DIM), lambda i: (_Z, _Z, _Z)),
        ],
        out_specs=[
            pl.BlockSpec((br, DIM), lambda i: (i, _Z)),
            pl.BlockSpec((br, N_NEIGHBOR), lambda i: (i, _Z)),
            pl.BlockSpec((br, N_NEIGHBOR, DIM), lambda i: (i, _Z, _Z)),
            pl.BlockSpec((br, N_NEIGHBOR), lambda i: (i, _Z)),
        ],
        out_shape=[
            jax.ShapeDtypeStruct((rows, DIM), jnp.float32),
            jax.ShapeDtypeStruct((rows, N_NEIGHBOR), jnp.float32),
            jax.ShapeDtypeStruct((rows, N_NEIGHBOR, DIM), jnp.float32),
            jax.ShapeDtypeStruct((rows, N_NEIGHBOR), jnp.int32),
        ],
    )(cl_emb, inv_b, nee_flat, nr_flat_f, ne_f, w_rel)


def _run_hop2(ue, nee_flat, nr_flat_f, ne_f, w_rel, br):
    rows = ue.shape[0]
    grid = (rows // br,)
    return pl.pallas_call(
        _hop2_body,
        grid=grid,
        in_specs=[
            pl.BlockSpec((br, DIM), lambda i: (i, _Z)),
            pl.BlockSpec((br * N_OLD, DIM), lambda i: (i, _Z)),
            pl.BlockSpec((br * N_OLD, 1), lambda i: (i, _Z)),
            pl.BlockSpec((br, N_OLD), lambda i: (i, _Z)),
            pl.BlockSpec((4, DIM, DIM), lambda i: (_Z, _Z, _Z)),
        ],
        out_specs=[
            pl.BlockSpec((br, N_NEIGHBOR), lambda i: (i, _Z)),
            pl.BlockSpec((br, N_NEIGHBOR, DIM), lambda i: (i, _Z, _Z)),
            pl.BlockSpec((br, N_NEIGHBOR), lambda i: (i, _Z)),
        ],
        out_shape=[
            jax.ShapeDtypeStruct((rows, N_NEIGHBOR), jnp.float32),
            jax.ShapeDtypeStruct((rows, N_NEIGHBOR, DIM), jnp.float32),
            jax.ShapeDtypeStruct((rows, N_NEIGHBOR), jnp.int32),
        ],
    )(ue, nee_flat, nr_flat_f, ne_f, w_rel)


def _agg_body(ue_ref, nee0_ref, su1_ref, su1f_ref, l1f_ref, r1f_ref,
              nee1tf_ref, l2_ref, su2_ref, nee2t_ref, scores_ref, pred_ref):
    bc = ue_ref.shape[0]
    su2 = su2_ref[...]
    nee2t = nee2t_ref[...]
    n1 = jnp.sum(nee2t * su2[:, :, None], axis=1) * 0.125
    nee1tf = nee1tf_ref[...]
    emb1p = jnp.maximum(n1 + nee1tf, 0.0)
    agg1 = jnp.sum(su2 * l2_ref[...], axis=-1, keepdims=True)
    r1f = r1f_ref[...]
    lab1p = r1f * l1f_ref[...] + (1.0 - r1f) * agg1

    su1 = su1_ref[...]
    nee1t = nee1tf.reshape(bc, N_NEIGHBOR, DIM)
    n0 = jnp.sum(nee1t * su1[:, :, None], axis=1) * 0.125
    emb0p = jnp.maximum(n0 + nee0_ref[...], 0.0)

    su1f = su1f_ref[...]
    wgt = emb1p * su1f
    n0p = jnp.sum(wgt.reshape(bc, N_NEIGHBOR, DIM), axis=1) * 0.125
    emb0pp = jnp.tanh(n0p + emb0p)
    lab0 = jnp.sum((su1f * lab1p).reshape(bc, N_NEIGHBOR, 1), axis=1)

    ue = ue_ref[...]
    dot = jnp.sum(ue * emb0pp, axis=-1, keepdims=True)
    scores_ref[...] = 1.0 / (1.0 + jnp.exp(-dot))
    pred_ref[...] = 1.0 / (1.0 + jnp.exp(-(lab0 - 0.5)))


def _run_agg(ue, nee0, su1, r1f, nee1t_flat, l1f, l2, su2, nee2t, bc):
    bs = ue.shape[0]
    su1f = su1.reshape(bs * N_NEIGHBOR, 1)
    grid = (bs // bc,)
    b8 = bc * N_NEIGHBOR
    return pl.pallas_call(
        _agg_body,
        grid=grid,
        in_specs=[
            pl.BlockSpec((bc, DIM), lambda i: (i, _Z)),
            pl.BlockSpec((bc, DIM), lambda i: (i, _Z)),
            pl.BlockSpec((bc, N_NEIGHBOR), lambda i: (i, _Z)),
            pl.BlockSpec((b8, 1), lambda i: (i, _Z)),
            pl.BlockSpec((b8, 1), lambda i: (i, _Z)),
            pl.BlockSpec((b8, 1), lambda i: (i, _Z)),
            pl.BlockSpec((b8, DIM), lambda i: (i, _Z)),
            pl.BlockSpec((b8, N_NEIGHBOR), lambda i: (i, _Z)),
            pl.BlockSpec((b8, N_NEIGHBOR), lambda i: (i, _Z)),
            pl.BlockSpec((b8, N_NEIGHBOR, DIM), lambda i: (i, _Z, _Z)),
        ],
        out_specs=[
            pl.BlockSpec((bc, 1), lambda i: (i, _Z)),
            pl.BlockSpec((bc, 1), lambda i: (i, _Z)),
        ],
        out_shape=[
            jax.ShapeDtypeStruct((bs, 1), jnp.float32),
            jax.ShapeDtypeStruct((bs, 1), jnp.float32),
        ],
    )(ue, nee0, su1, su1f, l1f, r1f, nee1t_flat, l2, su2, nee2t)


def _lookup(keys):
    h = (keys.astype(jnp.int64) * jnp.int64(2654435761)) % jnp.int64(2147483647)
    h = jnp.abs(h)
    known = (h % 7) == 0
    label = ((h // 7) % 2).astype(jnp.float32)
    return jnp.where(known, label, jnp.float32(0.5))


def kernel(user_indices, item_indices, entity_emb, W_relation,
           adj_entity, adj_relation, items_clicked, num_clicked):
    bs = user_indices.shape[0]
    n_pad_row = entity_emb.shape[0] - 1

    clicked = items_clicked[user_indices]
    clicked_pad = jnp.concatenate(
        [clicked, jnp.full((bs, HIST_PAD - clicked.shape[1]),
                           n_pad_row, dtype=jnp.int32)], axis=1)
    clicked_flat = clicked_pad.reshape(-1)
    inv = (1.0 / num_clicked[user_indices]).astype(jnp.float32)
    inv_b = jnp.broadcast_to(inv, (bs, DIM))
    item_idx32 = item_indices.astype(jnp.int32)

    g2 = _make_hop_gather_kernel(bs, True, 1024, clicked_n=bs * HIST_PAD)
    ne1, nr1, nee1_flat, nee0, cl_rows = g2(
        item_idx32, adj_entity, adj_relation, entity_emb, clicked_flat)
    cl_emb = cl_rows.reshape(bs, HIST_PAD, DIM)

    ue, su1, nee1t, ne1_top = _run_hop1(
        cl_emb, inv_b, nee1_flat,
        nr1.reshape(bs * N_OLD, 1).astype(jnp.float32),
        ne1.astype(jnp.float32), W_relation, 128)

    g3 = _make_hop_gather_kernel(bs * N_NEIGHBOR, False, 2048)
    ne2, nr2, nee2_flat = g3(ne1_top.reshape(-1), adj_entity, adj_relation,
                             entity_emb)

    ue_rep = jnp.repeat(ue, N_NEIGHBOR, axis=0)
    su2, nee2t, ne2_top = _run_hop2(
        ue_rep, nee2_flat,
        nr2.reshape(bs * N_NEIGHBOR * N_OLD, 1).astype(jnp.float32),
        ne2.astype(jnp.float32), W_relation, 128)

    offset = jnp.int64(entity_emb.shape[0])
    users = user_indices.reshape(bs, 1).astype(jnp.int64)
    key0 = users * offset + item_indices.reshape(bs, 1).astype(jnp.int64)
    key1 = users * offset + ne1_top.astype(jnp.int64)
    lab1_raw = _lookup(key1)
    hm1 = (key0 - key1) != 0
    r1 = ((lab1_raw - 0.5) != 0) & hm1
    l1 = (hm1.astype(jnp.float32) * lab1_raw
          + (~hm1).astype(jnp.float32) * 0.5)
    key2 = users * offset + ne2_top.reshape(bs, -1).astype(jnp.int64)
    lab2_raw = _lookup(key2)
    hm2 = (key0 - key2) != 0
    l2 = (hm2.astype(jnp.float32) * lab2_raw
          + (~hm2).astype(jnp.float32) * 0.5)

    scores_col, pred_col = _run_agg(
        ue, nee0, su1,
        r1.reshape(bs * N_NEIGHBOR, 1).astype(jnp.float32),
        nee1t.reshape(bs * N_NEIGHBOR, DIM),
        l1.reshape(bs * N_NEIGHBOR, 1),
        l2.reshape(bs * N_NEIGHBOR, N_NEIGHBOR),
        su2, nee2t, 128)
    return scores_col.reshape(bs), pred_col.reshape(bs)


# speedup vs baseline: 1.0080x; 1.0080x over previous
import functools

import jax
import jax.numpy as jnp
import numpy as np
from jax import lax
from jax.experimental import pallas as pl
from jax.experimental.pallas import tpu as pltpu
from jax.experimental.pallas import tpu_sc as plsc

N_NEIGHBOR = 8
DIM = 32
N_OLD = 32
HIST_PAD = 56
NEG = np.float32(-9e15)
_Z = np.int32(0)


def _sc_mesh():
    return plsc.VectorSubcoreMesh(core_axis_name="c", subcore_axis_name="s")


def _wid():
    info = plsc.get_sparse_core_info()
    return (lax.axis_index("s") * np.int32(info.num_cores)
            + lax.axis_index("c"))


@functools.lru_cache(maxsize=None)
def _make_row_gather_kernel(n_idx):
    info = plsc.get_sparse_core_info()
    nw = info.num_cores * info.num_subcores
    b_per_w = n_idx // nw
    n_chunks = b_per_w // 128

    @functools.partial(
        pl.kernel, mesh=_sc_mesh(),
        compiler_params=pltpu.CompilerParams(use_tc_tiling_on_sc=False),
        out_type=jax.ShapeDtypeStruct((n_idx, DIM), jnp.float32),
        scratch_types=[
            pltpu.VMEM((b_per_w,), jnp.int32),
            pltpu.VMEM((b_per_w, DIM), jnp.float32),
            pltpu.SemaphoreType.DMA,
        ],
    )
    def k(table_hbm, idx_hbm, out_hbm, idx_v, ebuf, sem):
        base = _wid() * np.int32(b_per_w)
        pltpu.sync_copy(idx_hbm.at[pl.ds(base, b_per_w)], idx_v)
        copies = []
        for c in range(n_chunks):
            copies.append(pltpu.async_copy(
                table_hbm.at[idx_v.at[pl.ds(c * 128, 128)]],
                ebuf.at[pl.ds(c * 128, 128)], sem))
        for c in copies:
            c.wait()
        pltpu.sync_copy(ebuf, out_hbm.at[pl.ds(base, b_per_w)])

    return k


@functools.lru_cache(maxsize=None)
def _make_hop_gather_kernel(n_seed, with_seed_emb, wave_rows, clicked_n=0):
    info = plsc.get_sparse_core_info()
    nw = info.num_cores * info.num_subcores
    s_per_w = n_seed // nw
    rows_w = s_per_w * N_OLD
    adj_chunk = min(s_per_w, 128)
    adj_chunks = s_per_w // adj_chunk
    n_waves = max(1, rows_w // wave_rows)
    wrows = rows_w // n_waves
    w_chunks = wrows // 128

    outs = [
        jax.ShapeDtypeStruct((n_seed, N_OLD), jnp.int32),
        jax.ShapeDtypeStruct((n_seed, N_OLD), jnp.int32),
        jax.ShapeDtypeStruct((n_seed * N_OLD, DIM), jnp.float32),
    ]
    scratch = [
        pltpu.VMEM((s_per_w,), jnp.int32),
        pltpu.VMEM((s_per_w, N_OLD), jnp.int32),
        pltpu.VMEM((s_per_w, N_OLD), jnp.int32),
        pltpu.VMEM((wrows, DIM), jnp.float32),
        pltpu.SemaphoreType.DMA,
    ]
    if with_seed_emb:
        outs.append(jax.ShapeDtypeStruct((n_seed, DIM), jnp.float32))
        scratch.insert(3, pltpu.VMEM((s_per_w, DIM), jnp.float32))
    cl_per_w = clicked_n // nw
    cl_chunks = cl_per_w // 128
    if clicked_n:
        outs.append(jax.ShapeDtypeStruct((clicked_n, DIM), jnp.float32))
        scratch.append(pltpu.VMEM((cl_per_w,), jnp.int32))
        scratch.append(pltpu.VMEM((cl_per_w, DIM), jnp.float32))

    @functools.partial(
        pl.kernel, mesh=_sc_mesh(),
        compiler_params=pltpu.CompilerParams(use_tc_tiling_on_sc=False),
        out_type=tuple(outs), scratch_types=scratch,
    )
    def k(seed_hbm, adj_e_hbm, adj_r_hbm, emb_hbm, *rest):
        cl_idx_hbm = None
        if clicked_n:
            cl_idx_hbm, *refs = rest
        else:
            refs = rest
        cl_out = idx_cl = cl_buf = None
        if with_seed_emb and clicked_n:
            (ne_out, nr_out, nee_out, se_out, cl_out,
             idx_v, ne_v, nr_v, se_v, ebuf, sem, idx_cl, cl_buf) = refs
        elif with_seed_emb:
            (ne_out, nr_out, nee_out, se_out,
             idx_v, ne_v, nr_v, se_v, ebuf, sem) = refs
        else:
            (ne_out, nr_out, nee_out,
             idx_v, ne_v, nr_v, ebuf, sem) = refs
        w = _wid()
        base = w * np.int32(s_per_w)
        pltpu.sync_copy(seed_hbm.at[pl.ds(base, s_per_w)], idx_v)
        if clicked_n:
            cl_base = w * np.int32(cl_per_w)
            pltpu.sync_copy(cl_idx_hbm.at[pl.ds(cl_base, cl_per_w)], idx_cl)
        copies = []
        if clicked_n:
            for c in range(cl_chunks):
                copies.append(pltpu.async_copy(
                    emb_hbm.at[idx_cl.at[pl.ds(c * 128, 128)]],
                    cl_buf.at[pl.ds(c * 128, 128)], sem))
        for c in range(adj_chunks):
            sl = pl.ds(c * adj_chunk, adj_chunk)
            copies.append(pltpu.async_copy(
                adj_e_hbm.at[idx_v.at[sl]], ne_v.at[sl], sem))
            copies.append(pltpu.async_copy(
                adj_r_hbm.at[idx_v.at[sl]], nr_v.at[sl], sem))
        if with_seed_emb:
            copies.append(pltpu.async_copy(emb_hbm.at[idx_v], se_v, sem))
        for c in copies:
            c.wait()
        pltpu.sync_copy(ne_v, ne_out.at[pl.ds(base, s_per_w)])
        pltpu.sync_copy(nr_v, nr_out.at[pl.ds(base, s_per_w)])
        if with_seed_emb:
            pltpu.sync_copy(se_v, se_out.at[pl.ds(base, s_per_w)])
        if clicked_n:
            pltpu.sync_copy(cl_buf, cl_out.at[pl.ds(cl_base, cl_per_w)])

        nee_base = w * np.int32(rows_w)
        seeds_per_wave = wrows // N_OLD
        for v in range(n_waves):
            copies = []
            for s in range(seeds_per_wave):
                copies.append(pltpu.async_copy(
                    emb_hbm.at[ne_v.at[np.int32(v * seeds_per_wave + s)]],
                    ebuf.at[pl.ds(s * N_OLD, N_OLD)], sem))
            for c in copies:
                c.wait()
            pltpu.sync_copy(
                ebuf, nee_out.at[pl.ds(nee_base + np.int32(v * wrows),
                                       wrows)])

    return k


def _hop_core(ue, nee_ref, nrf_ref, nef_ref, w_ref,
              su_out, neetop_out, netop_out):
    br = ue.shape[0]
    x = nee_ref[...]
    nrf = nrf_ref[...]
    for r in range(4):
        wr = w_ref[r]
        t = jnp.tanh(lax.dot_general(
            x, wr, (((1,), (1,)), ((), ())),
            preferred_element_type=jnp.float32))
        x = jnp.where(nrf == np.float32(r), t, x)
    neet = x.reshape(br, N_OLD, DIM)
    su = jnp.sum(neet * ue[:, None, :], axis=-1) * (1.0 / DIM)
    su = jnp.where(su == 0.0, NEG, su)
    m = jnp.max(su, axis=-1, keepdims=True)
    e = jnp.exp(su - m)
    su = e / jnp.sum(e, axis=-1, keepdims=True)

    iota = lax.broadcasted_iota(jnp.int32, (br, N_OLD), 1).astype(jnp.float32)
    nef = nef_ref[...]
    cur = su
    vals, ids, embs = [], [], []
    for _ in range(N_NEIGHBOR):
        mk = jnp.max(cur, axis=-1, keepdims=True)
        cand = jnp.where(cur == mk, iota, np.float32(N_OLD))
        sel = jnp.min(cand, axis=-1, keepdims=True)
        onehot = iota == sel
        ohf = onehot.astype(jnp.float32)
        vals.append(jnp.sum(su * ohf, -1, keepdims=True))
        ids.append(jnp.sum(nef * ohf, -1, keepdims=True))
        embs.append(jnp.sum(neet * ohf[:, :, None], axis=1)
                    .reshape(br, 1, DIM))
        cur = jnp.where(onehot, NEG, cur)

    st = jnp.concatenate(vals, axis=-1)
    st = jnp.where(st == 0.0, NEG, st)
    m2 = jnp.max(st, axis=-1, keepdims=True)
    e2 = jnp.exp(st - m2)
    su_out[...] = e2 / jnp.sum(e2, axis=-1, keepdims=True)
    netop_out[...] = jnp.concatenate(ids, axis=-1).astype(jnp.int32)
    neetop_out[...] = jnp.concatenate(embs, axis=1)


def _hop1_body(cl_ref, inv_ref, nee_ref, nrf_ref, nef_ref, w_ref,
               ue_out, su_out, neetop_out, netop_out):
    ue = jnp.sum(cl_ref[...], axis=1) * inv_ref[...]
    ue_out[...] = ue
    _hop_core(ue, nee_ref, nrf_ref, nef_ref, w_ref,
              su_out, neetop_out, netop_out)


def _hop2_body(ue_ref, nee_ref, nrf_ref, nef_ref, w_ref,
               su_out, neetop_out, netop_out):
    _hop_core(ue_ref[...], nee_ref, nrf_ref, nef_ref, w_ref,
              su_out, neetop_out, netop_out)


def _run_hop1(cl_emb, inv_b, nee_flat, nr_flat_f, ne_f, w_rel, br):
    rows = inv_b.shape[0]
    grid = (rows // br,)
    return pl.pallas_call(
        _hop1_body,
        grid=grid,
        in_specs=[
            pl.BlockSpec((br, HIST_PAD, DIM), lambda i: (i, _Z, _Z)),
            pl.BlockSpec((br, DIM), lambda i: (i, _Z)),
            pl.BlockSpec((br * N_OLD, DIM), lambda i: (i, _Z)),
            pl.BlockSpec((br * N_OLD, 1), lambda i: (i, _Z)),
            pl.BlockSpec((br, N_OLD), lambda i: (i, _Z)),
            pl.BlockSpec((4, DIM, DIM), lambda i: (_Z, _Z, _Z)),
        ],
        out_specs=[
            pl.BlockSpec((br, DIM), lambda i: (i, _Z)),
            pl.BlockSpec((br, N_NEIGHBOR), lambda i: (i, _Z)),
            pl.BlockSpec((br, N_NEIGHBOR, DIM), lambda i: (i, _Z, _Z)),
            pl.BlockSpec((br, N_NEIGHBOR), lambda i: (i, _Z)),
        ],
        out_shape=[
            jax.ShapeDtypeStruct((rows, DIM), jnp.float32),
            jax.ShapeDtypeStruct((rows, N_NEIGHBOR), jnp.float32),
            jax.ShapeDtypeStruct((rows, N_NEIGHBOR, DIM), jnp.float32),
            jax.ShapeDtypeStruct((rows, N_NEIGHBOR), jnp.int32),
        ],
    )(cl_emb, inv_b, nee_flat, nr_flat_f, ne_f, w_rel)


def _run_hop2(ue, nee_flat, nr_flat_f, ne_f, w_rel, br):
    rows = ue.shape[0]
    grid = (rows // br,)
    return pl.pallas_call(
        _hop2_body,
        grid=grid,
        in_specs=[
            pl.BlockSpec((br, DIM), lambda i: (i, _Z)),
            pl.BlockSpec((br * N_OLD, DIM), lambda i: (i, _Z)),
            pl.BlockSpec((br * N_OLD, 1), lambda i: (i, _Z)),
            pl.BlockSpec((br, N_OLD), lambda i: (i, _Z)),
            pl.BlockSpec((4, DIM, DIM), lambda i: (_Z, _Z, _Z)),
        ],
        out_specs=[
            pl.BlockSpec((br, N_NEIGHBOR), lambda i: (i, _Z)),
            pl.BlockSpec((br, N_NEIGHBOR, DIM), lambda i: (i, _Z, _Z)),
            pl.BlockSpec((br, N_NEIGHBOR), lambda i: (i, _Z)),
        ],
        out_shape=[
            jax.ShapeDtypeStruct((rows, N_NEIGHBOR), jnp.float32),
            jax.ShapeDtypeStruct((rows, N_NEIGHBOR, DIM), jnp.float32),
            jax.ShapeDtypeStruct((rows, N_NEIGHBOR), jnp.int32),
        ],
    )(ue, nee_flat, nr_flat_f, ne_f, w_rel)


def _agg_body(ue_ref, nee0_ref, su1_ref, su1f_ref, l1f_ref, r1f_ref,
              nee1tf_ref, l2_ref, su2_ref, nee2t_ref, scores_ref, pred_ref):
    bc = ue_ref.shape[0]
    su2 = su2_ref[...]
    nee2t = nee2t_ref[...]
    n1 = jnp.sum(nee2t * su2[:, :, None], axis=1) * 0.125
    nee1tf = nee1tf_ref[...]
    emb1p = jnp.maximum(n1 + nee1tf, 0.0)
    agg1 = jnp.sum(su2 * l2_ref[...], axis=-1, keepdims=True)
    r1f = r1f_ref[...]
    lab1p = r1f * l1f_ref[...] + (1.0 - r1f) * agg1

    su1 = su1_ref[...]
    nee1t = nee1tf.reshape(bc, N_NEIGHBOR, DIM)
    n0 = jnp.sum(nee1t * su1[:, :, None], axis=1) * 0.125
    emb0p = jnp.maximum(n0 + nee0_ref[...], 0.0)

    su1f = su1f_ref[...]
    wgt = emb1p * su1f
    n0p = jnp.sum(wgt.reshape(bc, N_NEIGHBOR, DIM), axis=1) * 0.125
    emb0pp = jnp.tanh(n0p + emb0p)
    lab0 = jnp.sum((su1f * lab1p).reshape(bc, N_NEIGHBOR, 1), axis=1)

    ue = ue_ref[...]
    dot = jnp.sum(ue * emb0pp, axis=-1, keepdims=True)
    scores_ref[...] = 1.0 / (1.0 + jnp.exp(-dot))
    pred_ref[...] = 1.0 / (1.0 + jnp.exp(-(lab0 - 0.5)))


def _run_agg(ue, nee0, su1, r1f, nee1t_flat, l1f, l2, su2, nee2t, bc):
    bs = ue.shape[0]
    su1f = su1.reshape(bs * N_NEIGHBOR, 1)
    grid = (bs // bc,)
    b8 = bc * N_NEIGHBOR
    return pl.pallas_call(
        _agg_body,
        grid=grid,
        in_specs=[
            pl.BlockSpec((bc, DIM), lambda i: (i, _Z)),
            pl.BlockSpec((bc, DIM), lambda i: (i, _Z)),
            pl.BlockSpec((bc, N_NEIGHBOR), lambda i: (i, _Z)),
            pl.BlockSpec((b8, 1), lambda i: (i, _Z)),
            pl.BlockSpec((b8, 1), lambda i: (i, _Z)),
            pl.BlockSpec((b8, 1), lambda i: (i, _Z)),
            pl.BlockSpec((b8, DIM), lambda i: (i, _Z)),
            pl.BlockSpec((b8, N_NEIGHBOR), lambda i: (i, _Z)),
            pl.BlockSpec((b8, N_NEIGHBOR), lambda i: (i, _Z)),
            pl.BlockSpec((b8, N_NEIGHBOR, DIM), lambda i: (i, _Z, _Z)),
        ],
        out_specs=[
            pl.BlockSpec((bc, 1), lambda i: (i, _Z)),
            pl.BlockSpec((bc, 1), lambda i: (i, _Z)),
        ],
        out_shape=[
            jax.ShapeDtypeStruct((bs, 1), jnp.float32),
            jax.ShapeDtypeStruct((bs, 1), jnp.float32),
        ],
    )(ue, nee0, su1, su1f, l1f, r1f, nee1t_flat, l2, su2, nee2t)


def _lookup(keys):
    h = (keys.astype(jnp.int64) * jnp.int64(2654435761)) % jnp.int64(2147483647)
    h = jnp.abs(h)
    known = (h % 7) == 0
    label = ((h // 7) % 2).astype(jnp.float32)
    return jnp.where(known, label, jnp.float32(0.5))


def kernel(user_indices, item_indices, entity_emb, W_relation,
           adj_entity, adj_relation, items_clicked, num_clicked):
    bs = user_indices.shape[0]
    n_pad_row = entity_emb.shape[0] - 1

    clicked = items_clicked[user_indices]
    clicked_pad = jnp.concatenate(
        [clicked, jnp.full((bs, HIST_PAD - clicked.shape[1]),
                           n_pad_row, dtype=jnp.int32)], axis=1)
    clicked_flat = clicked_pad.reshape(-1)
    inv = (1.0 / num_clicked[user_indices]).astype(jnp.float32)
    inv_b = jnp.broadcast_to(inv, (bs, DIM))
    item_idx32 = item_indices.astype(jnp.int32)

    cl_emb = _make_row_gather_kernel(bs * HIST_PAD)(
        entity_emb, clicked_flat).reshape(bs, HIST_PAD, DIM)

    g2 = _make_hop_gather_kernel(bs, True, 1024)
    ne1, nr1, nee1_flat, nee0 = g2(item_idx32, adj_entity, adj_relation,
                                   entity_emb)

    ue, su1, nee1t, ne1_top = _run_hop1(
        cl_emb, inv_b, nee1_flat,
        nr1.reshape(bs * N_OLD, 1).astype(jnp.float32),
        ne1.astype(jnp.float32), W_relation, 128)

    g3 = _make_hop_gather_kernel(bs * N_NEIGHBOR, False, 2048)
    ne2, nr2, nee2_flat = g3(ne1_top.reshape(-1), adj_entity, adj_relation,
                             entity_emb)

    ue_rep = jnp.repeat(ue, N_NEIGHBOR, axis=0)
    su2, nee2t, ne2_top = _run_hop2(
        ue_rep, nee2_flat,
        nr2.reshape(bs * N_NEIGHBOR * N_OLD, 1).astype(jnp.float32),
        ne2.astype(jnp.float32), W_relation, 128)

    offset = jnp.int64(entity_emb.shape[0])
    users = user_indices.reshape(bs, 1).astype(jnp.int64)
    key0 = users * offset + item_indices.reshape(bs, 1).astype(jnp.int64)
    key1 = users * offset + ne1_top.astype(jnp.int64)
    lab1_raw = _lookup(key1)
    hm1 = (key0 - key1) != 0
    r1 = ((lab1_raw - 0.5) != 0) & hm1
    l1 = (hm1.astype(jnp.float32) * lab1_raw
          + (~hm1).astype(jnp.float32) * 0.5)
    key2 = users * offset + ne2_top.reshape(bs, -1).astype(jnp.int64)
    lab2_raw = _lookup(key2)
    hm2 = (key0 - key2) != 0
    l2 = (hm2.astype(jnp.float32) * lab2_raw
          + (~hm2).astype(jnp.float32) * 0.5)

    scores_col, pred_col = _run_agg(
        ue, nee0, su1,
        r1.reshape(bs * N_NEIGHBOR, 1).astype(jnp.float32),
        nee1t.reshape(bs * N_NEIGHBOR, DIM),
        l1.reshape(bs * N_NEIGHBOR, 1),
        l2.reshape(bs * N_NEIGHBOR, N_NEIGHBOR),
        su2, nee2t, 128)
    return scores_col.reshape(bs), pred_col.reshape(bs)
